# baseline (speedup 1.0000x reference)
import functools

import jax
import jax.numpy as jnp
from jax import lax
from jax.experimental import pallas as pl
from jax.experimental.pallas import tpu as pltpu
from jax.experimental.pallas import tpu_sc as plsc

N = 8192
NW = 32
RPW = N // NW
LANES = 16
NCH = RPW // LANES
SUP = 8
_GATHER_DNUMS = lax.GatherDimensionNumbers(
    offset_dims=(), collapsed_slice_dims=(0,), start_index_map=(0,))


def _lane_bcast(vec, iv):
    return lax.gather(vec, iv[:, None], _GATHER_DNUMS, (1,),
                      mode=lax.GatherScatterMode.PROMISE_IN_BOUNDS)


def _bf16_round(x):
    u = lax.bitcast_convert_type(x, jnp.uint32)
    r = u + jnp.uint32(0x7FFF) + ((u >> jnp.uint32(16)) & jnp.uint32(1))
    r = r & jnp.uint32(0xFFFF0000)
    return lax.bitcast_convert_type(r, jnp.float32)


def _fill_r(xv, yv, zv, rv):
    def body(k, carry):
        b = k * LANES
        xs = xv[pl.ds(b, LANES)]
        ys = yv[pl.ds(b, LANES)]
        zs = zv[pl.ds(b, LANES)]
        rv[pl.ds(b, LANES)] = xs * xs + ys * ys + zs * zs
        xv[pl.ds(b, LANES)] = _bf16_round(xs)
        yv[pl.ds(b, LANES)] = _bf16_round(ys)
        zv[pl.ds(b, LANES)] = _bf16_round(zs)
        return carry
    lax.fori_loop(0, N // LANES, body, 0)


_LANE_IOTA = None


def _scan_pass(cx, cy, cz, cr, rx, ry, rz, rr, out_v, cm_v, tmp_v, row0):
    inf_v = jnp.full((LANES,), jnp.inf, jnp.float32)

    def init_body(k, carry):
        cm_v[pl.ds(k * LANES, LANES)] = inf_v
        return carry
    lax.fori_loop(0, N // LANES, init_body, 0)

    lane_ids = lax.iota(jnp.int32, LANES)

    for s in range(NCH // SUP):
        rows = []
        rnorm = []
        for c in range(SUP):
            b = row0 + (s * SUP + c) * LANES
            rows.append((rx[pl.ds(b, LANES)] * -2.0,
                         ry[pl.ds(b, LANES)] * -2.0,
                         rz[pl.ds(b, LANES)] * -2.0))
            rnorm.append(rr[pl.ds(b, LANES)])

        def body(jb, ms):
            b = jb * LANES
            cxv = cx[pl.ds(b, LANES)]
            cyv = cy[pl.ds(b, LANES)]
            czv = cz[pl.ds(b, LANES)]
            crv = cr[pl.ds(b, LANES)]

            def lane_body(l, carry):
                ms2 = carry[:SUP]
                cmv = carry[SUP]
                iv = jnp.full((LANES,), l, jnp.int32)
                bx = _lane_bcast(cxv, iv)
                by = _lane_bcast(cyv, iv)
                bz = _lane_bcast(czv, iv)
                bc = _lane_bcast(crv, iv)
                out = []
                full = []
                for c in range(SUP):
                    xs, ys, zs = rows[c]
                    vf = rnorm[c] + xs * bx + ys * by + zs * bz + bc
                    out.append(jnp.minimum(ms2[c], vf))
                    full.append(vf)
                while len(full) > 1:
                    full = [jnp.minimum(full[i], full[i + 1])
                            for i in range(0, len(full), 2)]
                hvv = full[0]
                for k in (8, 4, 2, 1):
                    hvv = jnp.minimum(hvv, _lane_bcast(hvv, lane_ids ^ k))
                cmv = jnp.minimum(
                    cmv, jnp.where(lane_ids == iv, hvv, jnp.inf))
                return tuple(out) + (cmv,)

            carry = lax.fori_loop(0, LANES, lane_body,
                                  tuple(ms) + (inf_v,), unroll=2)
            cm_v[pl.ds(b, LANES)] = jnp.minimum(cm_v[pl.ds(b, LANES)],
                                                carry[SUP])
            return carry[:SUP]

        init = tuple(inf_v for _ in range(SUP))
        ms = lax.fori_loop(0, N // LANES, body, init)
        for c in range(SUP):
            out_v[pl.ds((s * SUP + c) * LANES, LANES)] = ms[c]


@functools.partial(
    pl.kernel,
    mesh=plsc.VectorSubcoreMesh(core_axis_name="c", subcore_axis_name="s"),
    out_type=[jax.ShapeDtypeStruct((N,), jnp.float32),
              jax.ShapeDtypeStruct((NW * N,), jnp.float32)],
    scratch_types=[pltpu.VMEM((N,), jnp.float32) for _ in range(9)]
                  + [pltpu.VMEM((RPW,), jnp.float32) for _ in range(2)],
)
def _chamfer_sc(px_hbm, py_hbm, pz_hbm, tx_hbm, ty_hbm, tz_hbm,
                fwd_hbm, cm_hbm,
                px, py, pz, tx, ty, tz, rp, rt, cmv, outf, tmpv):
    wid = lax.axis_index("s") * 2 + lax.axis_index("c")
    base = wid * RPW
    pltpu.sync_copy(px_hbm, px)
    pltpu.sync_copy(py_hbm, py)
    pltpu.sync_copy(pz_hbm, pz)
    pltpu.sync_copy(tx_hbm, tx)
    pltpu.sync_copy(ty_hbm, ty)
    pltpu.sync_copy(tz_hbm, tz)
    _fill_r(px, py, pz, rp)
    _fill_r(tx, ty, tz, rt)
    _scan_pass(tx, ty, tz, rt, px, py, pz, rp, outf, cmv, tmpv, base)
    pltpu.sync_copy(outf, fwd_hbm.at[pl.ds(base, RPW)])
    pltpu.sync_copy(cmv, cm_hbm.at[pl.ds(wid * N, N)])


def _finish_body(f_ref, cm_ref, o_ref):
    f = f_ref[...]
    b = jnp.min(cm_ref[...], axis=0, keepdims=True)
    s1 = jnp.sum(jnp.sqrt(jnp.maximum(f, 1e-12)))
    s2 = jnp.sum(jnp.sqrt(jnp.maximum(b, 1e-12)))
    o_ref[...] = jnp.full((1, 1), (s1 + s2) * (1.0 / N), jnp.float32)


def kernel(predicted_set, target_set):
    pt = predicted_set.T
    tt = target_set.T
    fwd2, cmflat = _chamfer_sc(pt[0], pt[1], pt[2], tt[0], tt[1], tt[2])
    out = pl.pallas_call(
        _finish_body,
        out_shape=jax.ShapeDtypeStruct((1, 1), jnp.float32),
    )(fwd2.reshape(64, 128), cmflat.reshape(NW, N))
    return out[0, 0]


# speedup vs baseline: 1.0295x; 1.0295x over previous
import functools

import jax
import jax.numpy as jnp
from jax import lax
from jax.experimental import pallas as pl
from jax.experimental.pallas import tpu as pltpu
from jax.experimental.pallas import tpu_sc as plsc

N = 8192
NW = 32
RPW = N // NW
LANES = 16
NCH = RPW // LANES
SUP = 8
_GATHER_DNUMS = lax.GatherDimensionNumbers(
    offset_dims=(), collapsed_slice_dims=(0,), start_index_map=(0,))


def _lane_bcast(vec, iv):
    return lax.gather(vec, iv[:, None], _GATHER_DNUMS, (1,),
                      mode=lax.GatherScatterMode.PROMISE_IN_BOUNDS)


def _bf16_round(x):
    u = lax.bitcast_convert_type(x, jnp.uint32)
    r = u + jnp.uint32(0x7FFF) + ((u >> jnp.uint32(16)) & jnp.uint32(1))
    r = r & jnp.uint32(0xFFFF0000)
    return lax.bitcast_convert_type(r, jnp.float32)


def _fill_r(xv, yv, zv, rv):
    def body(k, carry):
        b = k * LANES
        xs = xv[pl.ds(b, LANES)]
        ys = yv[pl.ds(b, LANES)]
        zs = zv[pl.ds(b, LANES)]
        rv[pl.ds(b, LANES)] = xs * xs + ys * ys + zs * zs
        xv[pl.ds(b, LANES)] = _bf16_round(xs)
        yv[pl.ds(b, LANES)] = _bf16_round(ys)
        zv[pl.ds(b, LANES)] = _bf16_round(zs)
        return carry
    lax.fori_loop(0, N // LANES, body, 0)


def _scan_pass(cx, cy, cz, cr, rx, ry, rz, rr, out_v, cm_v, row0):
    inf_v = jnp.full((LANES,), jnp.inf, jnp.float32)

    def init_body(k, carry):
        cm_v[pl.ds(k * LANES, LANES)] = inf_v
        return carry
    lax.fori_loop(0, N // LANES, init_body, 0)

    lane_ids = lax.iota(jnp.int32, LANES)

    for s in range(NCH // SUP):
        rows = []
        rnorm = []
        for c in range(SUP):
            b = row0 + (s * SUP + c) * LANES
            rows.append((rx[pl.ds(b, LANES)] * -2.0,
                         ry[pl.ds(b, LANES)] * -2.0,
                         rz[pl.ds(b, LANES)] * -2.0))
            rnorm.append(rr[pl.ds(b, LANES)])

        def body(jb, ms):
            b = jb * LANES
            cxv = cx[pl.ds(b, LANES)]
            cyv = cy[pl.ds(b, LANES)]
            czv = cz[pl.ds(b, LANES)]
            crv = cr[pl.ds(b, LANES)]

            def lane_body(l, carry):
                ms2 = carry[:SUP]
                cmv = carry[SUP]
                iv = jnp.full((LANES,), l, jnp.int32)
                bx = _lane_bcast(cxv, iv)
                by = _lane_bcast(cyv, iv)
                bz = _lane_bcast(czv, iv)
                bc = _lane_bcast(crv, iv)
                out = []
                full = []
                for c in range(SUP):
                    xs, ys, zs = rows[c]
                    v = bc + xs * bx + ys * by + zs * bz
                    out.append(jnp.minimum(ms2[c], v))
                    full.append(v + rnorm[c])
                while len(full) > 1:
                    full = [jnp.minimum(full[i], full[i + 1])
                            for i in range(0, len(full), 2)]
                hvv = full[0]
                for k in (8, 4, 2, 1):
                    hvv = jnp.minimum(hvv, _lane_bcast(hvv, lane_ids ^ k))
                cmv = jnp.minimum(
                    cmv, jnp.where(lane_ids == iv, hvv, jnp.inf))
                return tuple(out) + (cmv,)

            carry = lax.fori_loop(0, LANES, lane_body,
                                  tuple(ms) + (inf_v,), unroll=2)
            cm_v[pl.ds(b, LANES)] = jnp.minimum(cm_v[pl.ds(b, LANES)],
                                                carry[SUP])
            return carry[:SUP]

        init = tuple(inf_v for _ in range(SUP))
        ms = lax.fori_loop(0, N // LANES, body, init)
        for c in range(SUP):
            ri = rr[pl.ds(row0 + (s * SUP + c) * LANES, LANES)]
            out_v[pl.ds((s * SUP + c) * LANES, LANES)] = ri + ms[c]


@functools.partial(
    pl.kernel,
    mesh=plsc.VectorSubcoreMesh(core_axis_name="c", subcore_axis_name="s"),
    out_type=[jax.ShapeDtypeStruct((N,), jnp.float32),
              jax.ShapeDtypeStruct((NW * N,), jnp.float32)],
    scratch_types=[pltpu.VMEM((N,), jnp.float32) for _ in range(9)]
                  + [pltpu.VMEM((RPW,), jnp.float32)],
)
def _chamfer_sc(px_hbm, py_hbm, pz_hbm, tx_hbm, ty_hbm, tz_hbm,
                fwd_hbm, cm_hbm,
                px, py, pz, tx, ty, tz, rp, rt, cmv, outf):
    wid = lax.axis_index("s") * 2 + lax.axis_index("c")
    base = wid * RPW
    pltpu.sync_copy(px_hbm, px)
    pltpu.sync_copy(py_hbm, py)
    pltpu.sync_copy(pz_hbm, pz)
    pltpu.sync_copy(tx_hbm, tx)
    pltpu.sync_copy(ty_hbm, ty)
    pltpu.sync_copy(tz_hbm, tz)
    _fill_r(px, py, pz, rp)
    _fill_r(tx, ty, tz, rt)
    _scan_pass(tx, ty, tz, rt, px, py, pz, rp, outf, cmv, base)
    pltpu.sync_copy(outf, fwd_hbm.at[pl.ds(base, RPW)])
    pltpu.sync_copy(cmv, cm_hbm.at[pl.ds(wid * N, N)])


def _finish_body(f_ref, cm_ref, o_ref):
    f = f_ref[...]
    b = jnp.min(cm_ref[...], axis=0, keepdims=True)
    s1 = jnp.sum(jnp.sqrt(jnp.maximum(f, 1e-12)))
    s2 = jnp.sum(jnp.sqrt(jnp.maximum(b, 1e-12)))
    o_ref[...] = jnp.full((1, 1), (s1 + s2) * (1.0 / N), jnp.float32)


def kernel(predicted_set, target_set):
    pt = predicted_set.T
    tt = target_set.T
    fwd2, cmflat = _chamfer_sc(pt[0], pt[1], pt[2], tt[0], tt[1], tt[2])
    out = pl.pallas_call(
        _finish_body,
        out_shape=jax.ShapeDtypeStruct((1, 1), jnp.float32),
    )(fwd2.reshape(64, 128), cmflat.reshape(NW, N))
    return out[0, 0]
